# baseline (speedup 1.0000x reference)
import functools

import jax
import jax.numpy as jnp
from jax import lax
from jax.experimental import pallas as pl
from jax.experimental.pallas import tpu as pltpu
from jax.experimental.pallas import tpu_sc as plsc

_NUM_ROWS = 16384
_NUM_COLS = 100
_D = 32
_NTOT = _NUM_ROWS * _NUM_COLS
_T128_ROWS = 250_000
_NC = 2
_NS = 16
_NW = _NC * _NS
_SLABS_PER_W = _NUM_ROWS // _NW
_CHUNK_LOOK = 400
_HALF = 200
_NCH = _SLABS_PER_W * _NUM_COLS // _CHUNK_LOOK
_NH = 2 * _NCH
_NVREG = _CHUNK_LOOK // 16


def _make_emb_kernel():
  mesh = plsc.VectorSubcoreMesh(core_axis_name="c", subcore_axis_name="s")

  @functools.partial(
      pl.kernel,
      out_type=jax.ShapeDtypeStruct((_NUM_ROWS, _NUM_COLS, _D), jnp.float32),
      mesh=mesh,
      scratch_types=[
          pltpu.VMEM((2 * _CHUNK_LOOK,), jnp.int32),
          pltpu.VMEM((2 * _CHUNK_LOOK,), jnp.int32),
          pltpu.VMEM((2 * _CHUNK_LOOK,), jnp.int32),
          pltpu.VMEM((2 * _HALF, 128), jnp.float32),
          pltpu.VMEM((2 * _HALF, _D), jnp.float32),
          pltpu.SemaphoreType.DMA((2,)),
          pltpu.SemaphoreType.DMA((2,)),
          pltpu.SemaphoreType.DMA((2,)),
      ],
      compiler_params=pltpu.CompilerParams(use_tc_tiling_on_sc=True,
                                           needs_layout_passes=False),
  )
  def emb(idx_hbm, t128_hbm, out_hbm, idxc_v, idx4_v, g_v, rows_v, stage_v,
          sem_i, sem_g, sem_o):
    wid = lax.axis_index("s") * _NC + lax.axis_index("c")
    slab0 = wid * _SLABS_PER_W
    p_base = slab0 * _NUM_COLS
    lane = lax.iota(jnp.int32, 16)

    def idx_copy(c):
      sl = lax.rem(c, 2)
      return pltpu.make_async_copy(
          idx_hbm.at[pl.ds(p_base + c * _CHUNK_LOOK, _CHUNK_LOOK)],
          idxc_v.at[pl.ds(sl * _CHUNK_LOOK, _CHUNK_LOOK)], sem_i.at[sl])

    def prep(c):
      off = lax.rem(c, 2) * _CHUNK_LOOK

      @plsc.parallel_loop(0, _NVREG, unroll=5)
      def _body(v):
        x = idxc_v[pl.ds(off + v * 16, 16)]
        idx4_v[pl.ds(off + v * 16, 16)] = lax.shift_right_logical(x, 2)
        g_v[pl.ds(off + v * 16, 16)] = lax.bitwise_and(x, 3)

    def gather_copy(h):
      sl = lax.rem(h, 2)
      ioff = lax.rem(lax.div(h, 2), 2) * _CHUNK_LOOK + lax.rem(h, 2) * _HALF
      return pltpu.make_async_copy(
          t128_hbm.at[idx4_v.at[pl.ds(ioff, _HALF)]],
          rows_v.at[pl.ds(sl * _HALF, _HALF)], sem_g.at[sl])

    def write_copy(h, s):
      sl = lax.rem(h, 2)
      return pltpu.make_async_copy(
          stage_v.at[pl.ds(sl * _HALF + s * _NUM_COLS, _NUM_COLS), :],
          out_hbm.at[slab0 + h * 2 + s], sem_o.at[sl])

    def select(h):
      sl = lax.rem(h, 2)
      goff = lax.rem(lax.div(h, 2), 2) * _CHUNK_LOOK + sl * _HALF
      roff = sl * _HALF

      @plsc.parallel_loop(0, _HALF, unroll=4)
      def _body(n):
        nn = jnp.full((16,), n, dtype=jnp.int32)
        g = plsc.load_gather(g_v, [nn + goff])
        col0 = g * 32 + lane
        stage_v[roff + n, pl.ds(0, 16)] = plsc.load_gather(
            rows_v, [nn + roff, col0])
        stage_v[roff + n, pl.ds(16, 16)] = plsc.load_gather(
            rows_v, [nn + roff, col0 + 16])

    idx_copy(0).start()
    idx_copy(0).wait()
    prep(0)
    idx_copy(1).start()
    gather_copy(0).start()

    def step(h, carry):
      @pl.when(jnp.logical_and(h + 1 < _NH, lax.rem(h, 2) == 1))
      def _():
        c2 = lax.div(h + 1, 2)
        idx_copy(c2).wait()
        prep(c2)

        @pl.when(c2 + 1 < _NCH)
        def _():
          idx_copy(c2 + 1).start()

      @pl.when(h + 1 < _NH)
      def _():
        gather_copy(h + 1).start()

      gather_copy(h).wait()

      @pl.when(h >= 2)
      def _():
        write_copy(h - 2, 0).wait()
        write_copy(h - 2, 1).wait()

      write_copy(h, 0).start()
      write_copy(h, 1).start()
      return carry

    lax.fori_loop(0, _NH, step, 0)

    write_copy(_NH - 2, 0).wait()
    write_copy(_NH - 2, 1).wait()
    write_copy(_NH - 1, 0).wait()
    write_copy(_NH - 1, 1).wait()

  return emb


_emb = _make_emb_kernel()


@jax.jit
def kernel(group_indices, embedding):
  flat = group_indices.reshape(_NTOT)
  t128 = embedding.reshape(_T128_ROWS, 128)
  return _emb(flat, t128)


# speedup vs baseline: 1.0005x; 1.0005x over previous
import functools

import jax
import jax.numpy as jnp
from jax import lax
from jax.experimental import pallas as pl
from jax.experimental.pallas import tpu as pltpu
from jax.experimental.pallas import tpu_sc as plsc

_NUM_ROWS = 16384
_NUM_COLS = 100
_D = 32
_NTOT = _NUM_ROWS * _NUM_COLS
_T128_ROWS = 250_000
_NC = 2
_NS = 16
_NW = _NC * _NS
_SLABS_PER_W = _NUM_ROWS // _NW
_CHUNK_LOOK = 400
_HALF = 200
_NCH = _SLABS_PER_W * _NUM_COLS // _CHUNK_LOOK
_NH = 2 * _NCH
_NVREG = _CHUNK_LOOK // 16


def _make_emb_kernel():
  mesh = plsc.VectorSubcoreMesh(core_axis_name="c", subcore_axis_name="s")

  @functools.partial(
      pl.kernel,
      out_type=jax.ShapeDtypeStruct((_NUM_ROWS, _NUM_COLS, _D), jnp.float32),
      mesh=mesh,
      scratch_types=[
          pltpu.VMEM((2 * _CHUNK_LOOK,), jnp.int32),
          pltpu.VMEM((2 * _CHUNK_LOOK,), jnp.int32),
          pltpu.VMEM((2 * _CHUNK_LOOK,), jnp.int32),
          pltpu.VMEM((2 * _HALF, 128), jnp.float32),
          pltpu.VMEM((2 * _HALF, _D), jnp.float32),
          pltpu.SemaphoreType.DMA((2,)),
          pltpu.SemaphoreType.DMA((2,)),
          pltpu.SemaphoreType.DMA((2,)),
      ],
      compiler_params=pltpu.CompilerParams(use_tc_tiling_on_sc=True,
                                           needs_layout_passes=False),
  )
  def emb(idx_hbm, t128_hbm, out_hbm, idxc_v, idx4_v, g_v, rows_v, stage_v,
          sem_i, sem_g, sem_o):
    wid = lax.axis_index("s") * _NC + lax.axis_index("c")
    slab0 = wid * _SLABS_PER_W
    p_base = slab0 * _NUM_COLS
    lane = lax.iota(jnp.int32, 16)

    def idx_copy(c):
      sl = lax.rem(c, 2)
      return pltpu.make_async_copy(
          idx_hbm.at[pl.ds(p_base + c * _CHUNK_LOOK, _CHUNK_LOOK)],
          idxc_v.at[pl.ds(sl * _CHUNK_LOOK, _CHUNK_LOOK)], sem_i.at[sl])

    def prep(c):
      off = lax.rem(c, 2) * _CHUNK_LOOK

      @plsc.parallel_loop(0, _NVREG, unroll=5)
      def _body(v):
        x = idxc_v[pl.ds(off + v * 16, 16)]
        idx4_v[pl.ds(off + v * 16, 16)] = lax.shift_right_logical(x, 2)
        g_v[pl.ds(off + v * 16, 16)] = lax.bitwise_and(x, 3)

    def gather_copy(h):
      sl = lax.rem(h, 2)
      ioff = lax.rem(lax.div(h, 2), 2) * _CHUNK_LOOK + lax.rem(h, 2) * _HALF
      return pltpu.make_async_copy(
          t128_hbm.at[idx4_v.at[pl.ds(ioff, _HALF)]],
          rows_v.at[pl.ds(sl * _HALF, _HALF)], sem_g.at[sl])

    def write_copy(h, s):
      sl = lax.rem(h, 2)
      return pltpu.make_async_copy(
          stage_v.at[pl.ds(sl * _HALF + s * _NUM_COLS, _NUM_COLS), :],
          out_hbm.at[slab0 + h * 2 + s], sem_o.at[sl])

    def select(h):
      sl = lax.rem(h, 2)
      goff = lax.rem(lax.div(h, 2), 2) * _CHUNK_LOOK + sl * _HALF
      roff = sl * _HALF

      @plsc.parallel_loop(0, _HALF, unroll=4)
      def _body(n):
        nn = jnp.full((16,), n, dtype=jnp.int32)
        g = plsc.load_gather(g_v, [nn + goff])
        col0 = g * 32 + lane
        stage_v[roff + n, pl.ds(0, 16)] = plsc.load_gather(
            rows_v, [nn + roff, col0])
        stage_v[roff + n, pl.ds(16, 16)] = plsc.load_gather(
            rows_v, [nn + roff, col0 + 16])

    idx_copy(0).start()
    idx_copy(0).wait()
    prep(0)
    idx_copy(1).start()
    gather_copy(0).start()

    def step(h, carry):
      @pl.when(jnp.logical_and(h + 1 < _NH, lax.rem(h, 2) == 1))
      def _():
        c2 = lax.div(h + 1, 2)
        idx_copy(c2).wait()
        prep(c2)

        @pl.when(c2 + 1 < _NCH)
        def _():
          idx_copy(c2 + 1).start()

      @pl.when(h + 1 < _NH)
      def _():
        gather_copy(h + 1).start()

      gather_copy(h).wait()

      @pl.when(h >= 2)
      def _():
        write_copy(h - 2, 0).wait()
        write_copy(h - 2, 1).wait()

      select(h)
      write_copy(h, 0).start()
      write_copy(h, 1).start()
      return carry

    lax.fori_loop(0, _NH, step, 0)

    write_copy(_NH - 2, 0).wait()
    write_copy(_NH - 2, 1).wait()
    write_copy(_NH - 1, 0).wait()
    write_copy(_NH - 1, 1).wait()

  return emb


_emb = _make_emb_kernel()


@jax.jit
def kernel(group_indices, embedding):
  flat = group_indices.reshape(_NTOT)
  t128 = embedding.reshape(_T128_ROWS, 128)
  return _emb(flat, t128)


# speedup vs baseline: 1.3784x; 1.3778x over previous
import functools

import jax
import jax.numpy as jnp
from jax import lax
from jax.experimental import pallas as pl
from jax.experimental.pallas import tpu as pltpu
from jax.experimental.pallas import tpu_sc as plsc

_NUM_ROWS = 16384
_NUM_COLS = 100
_D = 32
_T128_ROWS = 250_000
_NC = 2
_NS = 16
_NW = _NC * _NS
_IPW = _NUM_ROWS // _NW
_SUB = 128
_NSUB = _IPW // _SUB
_NH = _NUM_COLS * _NSUB
_NB = _SUB // 16


def _make_emb_kernel():
  mesh = plsc.VectorSubcoreMesh(core_axis_name="c", subcore_axis_name="s")

  @functools.partial(
      pl.kernel,
      out_type=jax.ShapeDtypeStruct((_NUM_COLS, _D, _NUM_ROWS), jnp.float32),
      mesh=mesh,
      scratch_types=[
          pltpu.VMEM((2 * _IPW,), jnp.int32),
          pltpu.VMEM((2 * _IPW,), jnp.int32),
          pltpu.VMEM((2 * _IPW,), jnp.int32),
          pltpu.VMEM((2 * _SUB, 128), jnp.float32),
          pltpu.VMEM((2 * _D, 128), jnp.float32),
          pltpu.SemaphoreType.DMA((2,)),
          pltpu.SemaphoreType.DMA((2,)),
          pltpu.SemaphoreType.DMA((2,)),
      ],
      compiler_params=pltpu.CompilerParams(use_tc_tiling_on_sc=True,
                                           needs_layout_passes=False),
  )
  def emb(idxt_hbm, t128_hbm, out_hbm, idxc_v, idx4_v, g_v, rows_v, stage_v,
          sem_i, sem_g, sem_o):
    wid = lax.axis_index("s") * _NC + lax.axis_index("c")
    i0 = wid * _IPW
    lane = lax.iota(jnp.int32, 16)

    def idx_copy(j):
      jr = lax.rem(j, 2)
      return pltpu.make_async_copy(
          idxt_hbm.at[j, pl.ds(i0, _IPW)],
          idxc_v.at[pl.ds(jr * _IPW, _IPW)], sem_i.at[jr])

    def prep(j):
      off = lax.rem(j, 2) * _IPW

      @plsc.parallel_loop(0, _IPW // 16, unroll=8)
      def _body(v):
        x = idxc_v[pl.ds(off + v * 16, 16)]
        idx4_v[pl.ds(off + v * 16, 16)] = lax.shift_right_logical(x, 2)
        g_v[pl.ds(off + v * 16, 16)] = lax.bitwise_and(x, 3)

    def gather_copy(h):
      sl = lax.rem(h, 2)
      ioff = lax.rem(lax.div(h, _NSUB), 2) * _IPW + lax.rem(h, _NSUB) * _SUB
      return pltpu.make_async_copy(
          t128_hbm.at[idx4_v.at[pl.ds(ioff, _SUB)]],
          rows_v.at[pl.ds(sl * _SUB, _SUB)], sem_g.at[sl])

    def write_copy(h):
      sl = lax.rem(h, 2)
      return pltpu.make_async_copy(
          stage_v.at[pl.ds(sl * _D, _D), :],
          out_hbm.at[lax.div(h, _NSUB), :,
                     pl.ds(i0 + lax.rem(h, _NSUB) * _SUB, _SUB)],
          sem_o.at[sl])

    def select(h):
      sl = lax.rem(h, 2)
      goff = lax.rem(lax.div(h, _NSUB), 2) * _IPW + lax.rem(h, _NSUB) * _SUB
      rows16 = [sl * _SUB + b * 16 + lane for b in range(_NB)]
      cols32 = [g_v[pl.ds(goff + b * 16, 16)] * 32 for b in range(_NB)]

      @plsc.parallel_loop(0, _D, unroll=2)
      def _body(f):
        fv = jnp.full((16,), f, dtype=jnp.int32)
        for b in range(_NB):
          stage_v[sl * _D + f, pl.ds(b * 16, 16)] = plsc.load_gather(
              rows_v, [rows16[b], cols32[b] + fv])

    idx_copy(0).start()
    idx_copy(0).wait()
    prep(0)
    idx_copy(1).start()
    gather_copy(0).start()

    def step(h, carry):
      @pl.when(jnp.logical_and(h + 1 < _NH, lax.rem(h, _NSUB) == _NSUB - 1))
      def _():
        j2 = lax.div(h + 1, _NSUB)
        idx_copy(j2).wait()
        prep(j2)

        @pl.when(j2 + 1 < _NUM_COLS)
        def _():
          idx_copy(j2 + 1).start()

      @pl.when(h + 1 < _NH)
      def _():
        gather_copy(h + 1).start()

      gather_copy(h).wait()

      @pl.when(h >= 2)
      def _():
        write_copy(h - 2).wait()

      select(h)
      write_copy(h).start()
      return carry

    lax.fori_loop(0, _NH, step, 0)

    write_copy(_NH - 2).wait()
    write_copy(_NH - 1).wait()

  return emb


_emb = _make_emb_kernel()


@jax.jit
def kernel(group_indices, embedding):
  idxt = group_indices.T
  t128 = embedding.reshape(_T128_ROWS, 128)
  res = _emb(idxt, t128)
  return jnp.transpose(res, (2, 0, 1))
